# Initial kernel scaffold; baseline (speedup 1.0000x reference)
import jax
import jax.numpy as jnp
from jax.experimental import pallas as pl


def kernel(x, edge_index, conv1_W, conv1_b, sk1_W, sk1_b, conv2_W, conv2_b, sk2_W, sk2_b, conv3_W, conv3_b, sk3_W, sk3_b, conv4_W, conv4_b, sk4_W, sk4_b, conv5_W, conv5_b, sk5_W, sk5_b, mix_W, mix_b, bn1_gamma, bn1_beta, bn1_mean, bn1_var, bn2_gamma, bn2_beta, bn2_mean, bn2_var, bn3_gamma, bn3_beta, bn3_mean, bn3_var, bn4_gamma, bn4_beta, bn4_mean, bn4_var, bn5_gamma, bn5_beta, bn5_mean, bn5_var):
    raise NotImplementedError("write your pallas kernel here")


# speedup vs baseline: 4.8290x; 4.8290x over previous
import functools

import jax
import jax.numpy as jnp
from jax import lax
from jax.experimental import pallas as pl
from jax.experimental.pallas import tpu as pltpu
from jax.experimental.pallas import tpu_sc as plsc

_N = 50000
_E = 800000
_NC, _NS = 2, 16
_CH = 128
_GRP = 4
_GE = _CH * _GRP
_EPAD = 819200
_NPAD = 50048
_RPT = _NPAD // _NS
_ZR = 136


def _seg_body(width, tab, srcq, dst3, out, idxs, idxd, rows, zbuf,
              si0, si1, sg0, sg1, ss0, ss1, acc):
    c = lax.axis_index("c")
    t = lax.axis_index("s")
    si = (si0, si1)
    sg = (sg0, sg1)
    ss = (ss0, ss1)

    def _zrow(r, _):
        for j0 in range(0, width - 15, 16):
            zbuf[r, j0:j0 + 16] = jnp.zeros((16,), jnp.float32)
        if width % 16:
            zbuf[r, width - 16:width] = jnp.zeros((16,), jnp.float32)
        return _
    lax.fori_loop(0, _ZR, _zrow, None)

    r0 = t * _RPT
    ept_rows = _EPAD // _NS // _CH
    niter = ept_rows // (2 * _GRP)

    for rnd in range(2):
        q = 2 * c + rnd
        for z in range(_RPT // _ZR):
            pltpu.async_copy(zbuf, acc.at[pl.ds(r0 + z * _ZR, _ZR)], sg0)
        for z in range(_RPT // _ZR):
            pltpu.make_async_copy(
                zbuf, acc.at[pl.ds(r0 + z * _ZR, _ZR)], sg0).wait()
        plsc.subcore_barrier()

        def _iter(i, _):
            @pl.when(i > 0)
            def _():
                for b in range(2):
                    for k in range(_GRP):
                        pltpu.make_async_copy(
                            rows.at[b, k], acc.at[idxd.at[b, k]], ss[b]).wait()

            row0 = t * ept_rows + i * 2 * _GRP
            hidx = []
            for b in range(2):
                hidx.append((
                    pltpu.async_copy(
                        srcq.at[q].at[pl.ds(row0 + b * _GRP, _GRP)],
                        idxs.at[b], si[b]),
                    pltpu.async_copy(
                        dst3.at[pl.ds(row0 + b * _GRP, _GRP)],
                        idxd.at[b], si[b]),
                ))
            hg = [[None] * _GRP for _ in range(2)]
            for b in range(2):
                for h in hidx[b]:
                    h.wait()
                for k in range(_GRP):
                    hg[b][k] = pltpu.async_copy(
                        tab.at[idxs.at[b, k]], rows.at[b, k], sg[b])
            for b in range(2):
                for k in range(_GRP):
                    hg[b][k].wait()
                    pltpu.async_copy(
                        rows.at[b, k], acc.at[idxd.at[b, k]], ss[b], add=True)
            return _

        lax.fori_loop(0, niter, _iter, None)
        for b in range(2):
            for k in range(_GRP):
                pltpu.make_async_copy(
                    rows.at[b, k], acc.at[idxd.at[b, k]], ss[b]).wait()
        plsc.subcore_barrier()
        pltpu.sync_copy(acc.at[pl.ds(r0, _RPT)], out.at[q].at[pl.ds(r0, _RPT)])
        plsc.subcore_barrier()


@functools.partial(jax.jit, static_argnums=(3,))
def _segsum(tab, srcq, dst3, width):
    mesh = plsc.VectorSubcoreMesh(core_axis_name="c", subcore_axis_name="s")
    kern = pl.kernel(
        functools.partial(_seg_body, width),
        out_type=jax.ShapeDtypeStruct((4, _NPAD, width), jnp.float32),
        mesh=mesh,
        compiler_params=pltpu.CompilerParams(use_tc_tiling_on_sc=False),
        scratch_types=[
            pltpu.VMEM((2, _GRP, _CH), jnp.int32),
            pltpu.VMEM((2, _GRP, _CH), jnp.int32),
            pltpu.VMEM((2, _GRP, _CH, width), jnp.float32),
            pltpu.VMEM((_ZR, width), jnp.float32),
            pltpu.SemaphoreType.DMA,
            pltpu.SemaphoreType.DMA,
            pltpu.SemaphoreType.DMA,
            pltpu.SemaphoreType.DMA,
            pltpu.SemaphoreType.DMA,
            pltpu.SemaphoreType.DMA,
            pltpu.VMEM_SHARED((_NPAD, width), jnp.float32),
        ],
    )
    return kern(tab, srcq, dst3)


def _deg_body(src3, out, idxd, ones, zbuf, si0, si1, ss0, ss1, acc):
    c = lax.axis_index("c")
    t = lax.axis_index("s")
    si = (si0, si1)
    ss = (ss0, ss1)

    def _orow(r, _):
        ones[r, 0:16] = jnp.ones((16,), jnp.float32)
        return _
    lax.fori_loop(0, _CH, _orow, None)

    def _zrow(r, _):
        zbuf[r, 0:16] = jnp.zeros((16,), jnp.float32)
        return _
    lax.fori_loop(0, _ZR, _zrow, None)
    r0 = t * _RPT
    for z in range(_RPT // _ZR):
        pltpu.async_copy(zbuf, acc.at[pl.ds(r0 + z * _ZR, _ZR)], ss0)
    for z in range(_RPT // _ZR):
        pltpu.make_async_copy(zbuf, acc.at[pl.ds(r0 + z * _ZR, _ZR)], ss0).wait()
    plsc.subcore_barrier()

    rows_per_tile = _EPAD // _CH // _NC // _NS
    niter = rows_per_tile // (2 * _GRP)

    def _iter(i, _):
        @pl.when(i > 0)
        def _():
            for b in range(2):
                for k in range(_GRP):
                    pltpu.make_async_copy(
                        ones, acc.at[idxd.at[b, k]], ss[b]).wait()
        row0 = (c * _NS + t) * rows_per_tile + i * 2 * _GRP
        h = []
        for b in range(2):
            h.append(pltpu.async_copy(
                src3.at[pl.ds(row0 + b * _GRP, _GRP)], idxd.at[b], si[b]))
        for b in range(2):
            h[b].wait()
            for k in range(_GRP):
                pltpu.async_copy(ones, acc.at[idxd.at[b, k]], ss[b], add=True)
        return _

    lax.fori_loop(0, niter, _iter, None)
    for b in range(2):
        for k in range(_GRP):
            pltpu.make_async_copy(ones, acc.at[idxd.at[b, k]], ss[b]).wait()
    plsc.subcore_barrier()
    pltpu.sync_copy(acc.at[pl.ds(r0, _RPT)], out.at[c].at[pl.ds(r0, _RPT)])


@jax.jit
def _degrees(src3):
    mesh = plsc.VectorSubcoreMesh(core_axis_name="c", subcore_axis_name="s")
    kern = pl.kernel(
        _deg_body,
        out_type=jax.ShapeDtypeStruct((_NC, _NPAD, 16), jnp.float32),
        mesh=mesh,
        compiler_params=pltpu.CompilerParams(use_tc_tiling_on_sc=False),
        scratch_types=[
            pltpu.VMEM((2, _GRP, _CH), jnp.int32),
            pltpu.VMEM((_CH, 16), jnp.float32),
            pltpu.VMEM((_ZR, 16), jnp.float32),
            pltpu.SemaphoreType.DMA,
            pltpu.SemaphoreType.DMA,
            pltpu.SemaphoreType.DMA,
            pltpu.SemaphoreType.DMA,
            pltpu.VMEM_SHARED((_NPAD, 16), jnp.float32),
        ],
    )
    return kern(src3)


def _mm_body(x_ref, w_ref, o_ref):
    o_ref[...] = jnp.dot(x_ref[...], w_ref[...],
                         preferred_element_type=jnp.float32)


def _matmul(x, w):
    m, k = x.shape
    _, n = w.shape
    bm = 512
    return pl.pallas_call(
        _mm_body,
        grid=(pl.cdiv(m, bm),),
        in_specs=[pl.BlockSpec((bm, k), lambda i: (i, 0)),
                  pl.BlockSpec((k, n), lambda i: (0, 0))],
        out_specs=pl.BlockSpec((bm, n), lambda i: (i, 0)),
        out_shape=jax.ShapeDtypeStruct((m, n), jnp.float32),
    )(x, w)


def _quarters(g, qw, width):
    g4 = g.reshape(_N, 4, qw).transpose(1, 0, 2)
    if width != qw:
        g4 = jnp.pad(g4, ((0, 0), (0, 0), (0, width - qw)))
    return g4.reshape(4 * _N, width)


def _unquarters(s4, qw):
    return s4[:, :_N, :qw].transpose(1, 0, 2).reshape(_N, 4 * qw)


def kernel(x, edge_index, conv1_W, conv1_b, sk1_W, sk1_b, conv2_W, conv2_b,
           sk2_W, sk2_b, conv3_W, conv3_b, sk3_W, sk3_b, conv4_W, conv4_b,
           sk4_W, sk4_b, conv5_W, conv5_b, sk5_W, sk5_b, mix_W, mix_b,
           bn1_gamma, bn1_beta, bn1_mean, bn1_var,
           bn2_gamma, bn2_beta, bn2_mean, bn2_var,
           bn3_gamma, bn3_beta, bn3_mean, bn3_var,
           bn4_gamma, bn4_beta, bn4_mean, bn4_var,
           bn5_gamma, bn5_beta, bn5_mean, bn5_var):
    src = edge_index[0]
    dst = edge_index[1]

    npad = _EPAD - _E
    pad_src = (jnp.arange(npad, dtype=jnp.int32) * 97) % _N
    pad_dst = _N + (jnp.arange(npad, dtype=jnp.int32) % (_NPAD - _N))
    src_p = jnp.concatenate([src, pad_src])
    dst_p = jnp.concatenate([dst, pad_dst])
    srcq = jnp.stack([src_p + q * _N for q in range(4)]
                     ).reshape(4, _EPAD // _CH, _CH)
    dst3 = dst_p.reshape(_EPAD // _CH, _CH)
    src3 = jnp.concatenate([src, pad_dst]).reshape(_EPAD // _CH, _CH)

    degp = _degrees(src3)
    deg = degp[0, :_N, 0] + degp[1, :_N, 0]
    dinv = jnp.where(deg > 0, lax.rsqrt(jnp.maximum(deg, 1.0)), 0.0)
    dinv = dinv[:, None]
    dinv2 = dinv * dinv

    def prop_raw(g, qw, width):
        s4 = _segsum(_quarters(g, qw, width), srcq, dst3, width)
        return _unquarters(s4, qw)

    params = dict(
        conv=[(conv1_W, conv1_b), (conv2_W, conv2_b), (conv3_W, conv3_b),
              (conv4_W, conv4_b), (conv5_W, conv5_b)],
        sk=[(sk1_W, sk1_b), (sk2_W, sk2_b), (sk3_W, sk3_b), (sk4_W, sk4_b),
            (sk5_W, sk5_b)],
        bn=[(bn1_gamma, bn1_beta, bn1_mean, bn1_var),
            (bn2_gamma, bn2_beta, bn2_mean, bn2_var),
            (bn3_gamma, bn3_beta, bn3_mean, bn3_var),
            (bn4_gamma, bn4_beta, bn4_mean, bn4_var),
            (bn5_gamma, bn5_beta, bn5_mean, bn5_var)],
    )

    h = x
    for i in range(5):
        W, b = params["conv"][i]
        skW, skb = params["sk"][i]
        gam, bet, mean, var = params["bn"][i]
        A = jnp.concatenate([4.0 * W[3], 2.0 * W[2], W[1] - 3.0 * W[3],
                             W[0] - W[2], skW[0]], axis=1)
        Y = _matmul(h, A)
        Y3, Y2, Y1, Y0, Ysk = (Y[:, :80], Y[:, 80:160], Y[:, 160:240],
                               Y[:, 240:320], Y[:, 320:400])
        s = prop_raw(dinv * Y3, 20, 24)
        s = prop_raw(dinv * Y2 - dinv2 * s, 20, 24)
        s = prop_raw(dinv * Y1 - dinv2 * s, 20, 24)
        pre = Y0 - dinv * s
        scale = gam * lax.rsqrt(var + 1e-5)
        shift = bet - mean * scale + b * scale
        h = jax.nn.relu(pre * scale + shift) + Ysk + skb

    hc = jnp.concatenate([h, x], axis=1)
    Am = jnp.concatenate([mix_W[1], mix_W[0]], axis=1)
    Ym = _matmul(hc, Am)
    g = jnp.pad(dinv * Ym[:, :48], ((0, 0), (0, 16)))
    s = prop_raw(g, 16, 16)[:, :48]
    return Ym[:, 48:96] + mix_b - dinv * s


# speedup vs baseline: 4.8583x; 1.0061x over previous
import functools

import jax
import jax.numpy as jnp
from jax import lax
from jax.experimental import pallas as pl
from jax.experimental.pallas import tpu as pltpu
from jax.experimental.pallas import tpu_sc as plsc

_N = 50000
_E = 800000
_NC, _NS = 2, 16
_CH = 320
_GRP = 2
_GE = _CH * _GRP
_EPAD = 819200
_NPAD = 50048
_RPT = _NPAD // _NS
_ZR = 136


def _seg_body(width, tab, srcq, dst3, out, idxs, idxd, rows, zbuf,
              si0, si1, sg0, sg1, ss0, ss1, acc):
    c = lax.axis_index("c")
    t = lax.axis_index("s")
    si = (si0, si1)
    sg = (sg0, sg1)
    ss = (ss0, ss1)

    def _zrow(r, _):
        for j0 in range(0, width - 15, 16):
            zbuf[r, j0:j0 + 16] = jnp.zeros((16,), jnp.float32)
        if width % 16:
            zbuf[r, width - 16:width] = jnp.zeros((16,), jnp.float32)
        return _
    lax.fori_loop(0, _ZR, _zrow, None)

    r0 = t * _RPT
    ept_rows = _EPAD // _NS // _CH
    niter = ept_rows // (2 * _GRP)

    for rnd in range(2):
        q = 2 * c + rnd
        for z in range(_RPT // _ZR):
            pltpu.async_copy(zbuf, acc.at[pl.ds(r0 + z * _ZR, _ZR)], sg0)
        for z in range(_RPT // _ZR):
            pltpu.make_async_copy(
                zbuf, acc.at[pl.ds(r0 + z * _ZR, _ZR)], sg0).wait()
        plsc.subcore_barrier()

        def _iter(i, _):
            @pl.when(i > 0)
            def _():
                for b in range(2):
                    for k in range(_GRP):
                        pltpu.make_async_copy(
                            rows.at[b, k], acc.at[idxd.at[b, k]], ss[b]).wait()

            row0 = t * ept_rows + i * 2 * _GRP
            hidx = []
            for b in range(2):
                hidx.append((
                    pltpu.async_copy(
                        srcq.at[q].at[pl.ds(row0 + b * _GRP, _GRP)],
                        idxs.at[b], si[b]),
                    pltpu.async_copy(
                        dst3.at[pl.ds(row0 + b * _GRP, _GRP)],
                        idxd.at[b], si[b]),
                ))
            hg = [[None] * _GRP for _ in range(2)]
            for b in range(2):
                for h in hidx[b]:
                    h.wait()
                for k in range(_GRP):
                    hg[b][k] = pltpu.async_copy(
                        tab.at[idxs.at[b, k]], rows.at[b, k], sg[b])
            for b in range(2):
                for k in range(_GRP):
                    hg[b][k].wait()
                    pltpu.async_copy(
                        rows.at[b, k], acc.at[idxd.at[b, k]], ss[b], add=True)
            return _

        lax.fori_loop(0, niter, _iter, None)
        for b in range(2):
            for k in range(_GRP):
                pltpu.make_async_copy(
                    rows.at[b, k], acc.at[idxd.at[b, k]], ss[b]).wait()
        plsc.subcore_barrier()
        pltpu.sync_copy(acc.at[pl.ds(r0, _RPT)], out.at[q].at[pl.ds(r0, _RPT)])
        plsc.subcore_barrier()


@functools.partial(jax.jit, static_argnums=(3,))
def _segsum(tab, srcq, dst3, width):
    mesh = plsc.VectorSubcoreMesh(core_axis_name="c", subcore_axis_name="s")
    kern = pl.kernel(
        functools.partial(_seg_body, width),
        out_type=jax.ShapeDtypeStruct((4, _NPAD, width), jnp.float32),
        mesh=mesh,
        compiler_params=pltpu.CompilerParams(use_tc_tiling_on_sc=False),
        scratch_types=[
            pltpu.VMEM((2, _GRP, _CH), jnp.int32),
            pltpu.VMEM((2, _GRP, _CH), jnp.int32),
            pltpu.VMEM((2, _GRP, _CH, width), jnp.float32),
            pltpu.VMEM((_ZR, width), jnp.float32),
            pltpu.SemaphoreType.DMA,
            pltpu.SemaphoreType.DMA,
            pltpu.SemaphoreType.DMA,
            pltpu.SemaphoreType.DMA,
            pltpu.SemaphoreType.DMA,
            pltpu.SemaphoreType.DMA,
            pltpu.VMEM_SHARED((_NPAD, width), jnp.float32),
        ],
    )
    return kern(tab, srcq, dst3)


def _deg_body(src3, out, idxd, ones, zbuf, si0, si1, ss0, ss1, acc):
    c = lax.axis_index("c")
    t = lax.axis_index("s")
    si = (si0, si1)
    ss = (ss0, ss1)

    def _orow(r, _):
        ones[r, 0:16] = jnp.ones((16,), jnp.float32)
        return _
    lax.fori_loop(0, _CH, _orow, None)

    def _zrow(r, _):
        zbuf[r, 0:16] = jnp.zeros((16,), jnp.float32)
        return _
    lax.fori_loop(0, _ZR, _zrow, None)
    r0 = t * _RPT
    for z in range(_RPT // _ZR):
        pltpu.async_copy(zbuf, acc.at[pl.ds(r0 + z * _ZR, _ZR)], ss0)
    for z in range(_RPT // _ZR):
        pltpu.make_async_copy(zbuf, acc.at[pl.ds(r0 + z * _ZR, _ZR)], ss0).wait()
    plsc.subcore_barrier()

    rows_per_tile = _EPAD // _CH // _NC // _NS
    niter = rows_per_tile // (2 * _GRP)

    def _iter(i, _):
        @pl.when(i > 0)
        def _():
            for b in range(2):
                for k in range(_GRP):
                    pltpu.make_async_copy(
                        ones, acc.at[idxd.at[b, k]], ss[b]).wait()
        row0 = (c * _NS + t) * rows_per_tile + i * 2 * _GRP
        h = []
        for b in range(2):
            h.append(pltpu.async_copy(
                src3.at[pl.ds(row0 + b * _GRP, _GRP)], idxd.at[b], si[b]))
        for b in range(2):
            h[b].wait()
            for k in range(_GRP):
                pltpu.async_copy(ones, acc.at[idxd.at[b, k]], ss[b], add=True)
        return _

    lax.fori_loop(0, niter, _iter, None)
    for b in range(2):
        for k in range(_GRP):
            pltpu.make_async_copy(ones, acc.at[idxd.at[b, k]], ss[b]).wait()
    plsc.subcore_barrier()
    pltpu.sync_copy(acc.at[pl.ds(r0, _RPT)], out.at[c].at[pl.ds(r0, _RPT)])


@jax.jit
def _degrees(src3):
    mesh = plsc.VectorSubcoreMesh(core_axis_name="c", subcore_axis_name="s")
    kern = pl.kernel(
        _deg_body,
        out_type=jax.ShapeDtypeStruct((_NC, _NPAD, 16), jnp.float32),
        mesh=mesh,
        compiler_params=pltpu.CompilerParams(use_tc_tiling_on_sc=False),
        scratch_types=[
            pltpu.VMEM((2, _GRP, _CH), jnp.int32),
            pltpu.VMEM((_CH, 16), jnp.float32),
            pltpu.VMEM((_ZR, 16), jnp.float32),
            pltpu.SemaphoreType.DMA,
            pltpu.SemaphoreType.DMA,
            pltpu.SemaphoreType.DMA,
            pltpu.SemaphoreType.DMA,
            pltpu.VMEM_SHARED((_NPAD, 16), jnp.float32),
        ],
    )
    return kern(src3)


def _mm_body(x_ref, w_ref, o_ref):
    o_ref[...] = jnp.dot(x_ref[...], w_ref[...],
                         preferred_element_type=jnp.float32)


def _matmul(x, w):
    m, k = x.shape
    _, n = w.shape
    bm = 512
    return pl.pallas_call(
        _mm_body,
        grid=(pl.cdiv(m, bm),),
        in_specs=[pl.BlockSpec((bm, k), lambda i: (i, 0)),
                  pl.BlockSpec((k, n), lambda i: (0, 0))],
        out_specs=pl.BlockSpec((bm, n), lambda i: (i, 0)),
        out_shape=jax.ShapeDtypeStruct((m, n), jnp.float32),
    )(x, w)


def _quarters(g, qw, width):
    g4 = g.reshape(_N, 4, qw).transpose(1, 0, 2)
    if width != qw:
        g4 = jnp.pad(g4, ((0, 0), (0, 0), (0, width - qw)))
    return g4.reshape(4 * _N, width)


def _unquarters(s4, qw):
    return s4[:, :_N, :qw].transpose(1, 0, 2).reshape(_N, 4 * qw)


def kernel(x, edge_index, conv1_W, conv1_b, sk1_W, sk1_b, conv2_W, conv2_b,
           sk2_W, sk2_b, conv3_W, conv3_b, sk3_W, sk3_b, conv4_W, conv4_b,
           sk4_W, sk4_b, conv5_W, conv5_b, sk5_W, sk5_b, mix_W, mix_b,
           bn1_gamma, bn1_beta, bn1_mean, bn1_var,
           bn2_gamma, bn2_beta, bn2_mean, bn2_var,
           bn3_gamma, bn3_beta, bn3_mean, bn3_var,
           bn4_gamma, bn4_beta, bn4_mean, bn4_var,
           bn5_gamma, bn5_beta, bn5_mean, bn5_var):
    src = edge_index[0]
    dst = edge_index[1]

    npad = _EPAD - _E
    pad_src = (jnp.arange(npad, dtype=jnp.int32) * 97) % _N
    pad_dst = _N + (jnp.arange(npad, dtype=jnp.int32) % (_NPAD - _N))
    src_p = jnp.concatenate([src, pad_src])
    dst_p = jnp.concatenate([dst, pad_dst])
    srcq = jnp.stack([src_p + q * _N for q in range(4)]
                     ).reshape(4, _EPAD // _CH, _CH)
    dst3 = dst_p.reshape(_EPAD // _CH, _CH)
    src3 = jnp.concatenate([src, pad_dst]).reshape(_EPAD // _CH, _CH)

    degp = _degrees(src3)
    deg = degp[0, :_N, 0] + degp[1, :_N, 0]
    dinv = jnp.where(deg > 0, lax.rsqrt(jnp.maximum(deg, 1.0)), 0.0)
    dinv = dinv[:, None]
    dinv2 = dinv * dinv

    def prop_raw(g, qw, width):
        s4 = _segsum(_quarters(g, qw, width), srcq, dst3, width)
        return _unquarters(s4, qw)

    params = dict(
        conv=[(conv1_W, conv1_b), (conv2_W, conv2_b), (conv3_W, conv3_b),
              (conv4_W, conv4_b), (conv5_W, conv5_b)],
        sk=[(sk1_W, sk1_b), (sk2_W, sk2_b), (sk3_W, sk3_b), (sk4_W, sk4_b),
            (sk5_W, sk5_b)],
        bn=[(bn1_gamma, bn1_beta, bn1_mean, bn1_var),
            (bn2_gamma, bn2_beta, bn2_mean, bn2_var),
            (bn3_gamma, bn3_beta, bn3_mean, bn3_var),
            (bn4_gamma, bn4_beta, bn4_mean, bn4_var),
            (bn5_gamma, bn5_beta, bn5_mean, bn5_var)],
    )

    h = x
    for i in range(5):
        W, b = params["conv"][i]
        skW, skb = params["sk"][i]
        gam, bet, mean, var = params["bn"][i]
        A = jnp.concatenate([4.0 * W[3], 2.0 * W[2], W[1] - 3.0 * W[3],
                             W[0] - W[2], skW[0]], axis=1)
        Y = _matmul(h, A)
        Y3, Y2, Y1, Y0, Ysk = (Y[:, :80], Y[:, 80:160], Y[:, 160:240],
                               Y[:, 240:320], Y[:, 320:400])
        s = prop_raw(dinv * Y3, 20, 24)
        s = prop_raw(dinv * Y2 - dinv2 * s, 20, 24)
        s = prop_raw(dinv * Y1 - dinv2 * s, 20, 24)
        pre = Y0 - dinv * s
        scale = gam * lax.rsqrt(var + 1e-5)
        shift = bet - mean * scale + b * scale
        h = jax.nn.relu(pre * scale + shift) + Ysk + skb

    hc = jnp.concatenate([h, x], axis=1)
    Am = jnp.concatenate([mix_W[1], mix_W[0]], axis=1)
    Ym = _matmul(hc, Am)
    g = jnp.pad(dinv * Ym[:, :48], ((0, 0), (0, 16)))
    s = prop_raw(g, 16, 16)[:, :48]
    return Ym[:, 48:96] + mix_b - dinv * s


# speedup vs baseline: 6.0480x; 1.2449x over previous
import functools

import jax
import jax.numpy as jnp
from jax import lax
from jax.experimental import pallas as pl
from jax.experimental.pallas import tpu as pltpu
from jax.experimental.pallas import tpu_sc as plsc

_N = 50000
_E = 800000
_NC, _NS = 2, 16
_CH = 320
_GRP = 2
_GE = _CH * _GRP
_EPAD = 819200
_NPAD = 50048
_RPT = _NPAD // _NS
_ZR = 136


def _seg_body(width, tab, srcq, dst3, out, idxs, idxd, rows, zbuf,
              si0, si1, sg0, sg1, ss0, ss1, acc):
    c = lax.axis_index("c")
    t = lax.axis_index("s")
    si = (si0, si1)
    sg = (sg0, sg1)
    ss = (ss0, ss1)

    def _zrow(r, _):
        for j0 in range(0, width - 15, 16):
            zbuf[r, j0:j0 + 16] = jnp.zeros((16,), jnp.float32)
        if width % 16:
            zbuf[r, width - 16:width] = jnp.zeros((16,), jnp.float32)
        return _
    lax.fori_loop(0, _ZR, _zrow, None)

    r0 = t * _RPT
    ept_rows = _EPAD // _NS // _CH
    niter = ept_rows // (2 * _GRP)

    for rnd in range(2):
        q = 2 * c + rnd
        for z in range(_RPT // _ZR):
            pltpu.async_copy(zbuf, acc.at[pl.ds(r0 + z * _ZR, _ZR)], sg0)
        for z in range(_RPT // _ZR):
            pltpu.make_async_copy(
                zbuf, acc.at[pl.ds(r0 + z * _ZR, _ZR)], sg0).wait()
        plsc.subcore_barrier()

        def _iter(i, _):
            @pl.when(i > 0)
            def _():
                for b in range(2):
                    for k in range(_GRP):
                        pltpu.make_async_copy(
                            rows.at[b, k], acc.at[idxd.at[b, k]], ss[b]).wait()

            row0 = t * ept_rows + i * 2 * _GRP
            hidx = []
            for b in range(2):
                hidx.append((
                    pltpu.async_copy(
                        srcq.at[q].at[pl.ds(row0 + b * _GRP, _GRP)],
                        idxs.at[b], si[b]),
                    pltpu.async_copy(
                        dst3.at[pl.ds(row0 + b * _GRP, _GRP)],
                        idxd.at[b], si[b]),
                ))
            hg = [[None] * _GRP for _ in range(2)]
            for b in range(2):
                for h in hidx[b]:
                    h.wait()
                for k in range(_GRP):
                    hg[b][k] = pltpu.async_copy(
                        tab.at[idxs.at[b, k]], rows.at[b, k], sg[b])
            for b in range(2):
                for k in range(_GRP):
                    hg[b][k].wait()
                    pltpu.async_copy(
                        rows.at[b, k], acc.at[idxd.at[b, k]], ss[b], add=True)
            return _

        lax.fori_loop(0, niter, _iter, None)
        for b in range(2):
            for k in range(_GRP):
                pltpu.make_async_copy(
                    rows.at[b, k], acc.at[idxd.at[b, k]], ss[b]).wait()
        plsc.subcore_barrier()
        pltpu.sync_copy(acc.at[pl.ds(r0, _RPT)], out.at[q].at[pl.ds(r0, _RPT)])
        plsc.subcore_barrier()


@functools.partial(jax.jit, static_argnums=(3,))
def _segsum(tab, srcq, dst3, width):
    mesh = plsc.VectorSubcoreMesh(core_axis_name="c", subcore_axis_name="s")
    kern = pl.kernel(
        functools.partial(_seg_body, width),
        out_type=jax.ShapeDtypeStruct((4, _NPAD, width), jnp.float32),
        mesh=mesh,
        compiler_params=pltpu.CompilerParams(use_tc_tiling_on_sc=False),
        scratch_types=[
            pltpu.VMEM((2, _GRP, _CH), jnp.int32),
            pltpu.VMEM((2, _GRP, _CH), jnp.int32),
            pltpu.VMEM((2, _GRP, _CH, width), jnp.float32),
            pltpu.VMEM((_ZR, width), jnp.float32),
            pltpu.SemaphoreType.DMA,
            pltpu.SemaphoreType.DMA,
            pltpu.SemaphoreType.DMA,
            pltpu.SemaphoreType.DMA,
            pltpu.SemaphoreType.DMA,
            pltpu.SemaphoreType.DMA,
            pltpu.VMEM_SHARED((_NPAD, width), jnp.float32),
        ],
    )
    return kern(tab, srcq, dst3)


def _deg_body(src3, out, idxd, ones, zbuf, si0, si1, ss0, ss1, acc):
    c = lax.axis_index("c")
    t = lax.axis_index("s")
    si = (si0, si1)
    ss = (ss0, ss1)

    def _orow(r, _):
        ones[r, 0:16] = jnp.ones((16,), jnp.float32)
        return _
    lax.fori_loop(0, _CH, _orow, None)

    def _zrow(r, _):
        zbuf[r, 0:16] = jnp.zeros((16,), jnp.float32)
        return _
    lax.fori_loop(0, _ZR, _zrow, None)
    r0 = t * _RPT
    for z in range(_RPT // _ZR):
        pltpu.async_copy(zbuf, acc.at[pl.ds(r0 + z * _ZR, _ZR)], ss0)
    for z in range(_RPT // _ZR):
        pltpu.make_async_copy(zbuf, acc.at[pl.ds(r0 + z * _ZR, _ZR)], ss0).wait()
    plsc.subcore_barrier()

    rows_per_tile = _EPAD // _CH // _NC // _NS
    niter = rows_per_tile // (2 * _GRP)

    def _iter(i, _):
        @pl.when(i > 0)
        def _():
            for b in range(2):
                for k in range(_GRP):
                    pltpu.make_async_copy(
                        ones, acc.at[idxd.at[b, k]], ss[b]).wait()
        row0 = (c * _NS + t) * rows_per_tile + i * 2 * _GRP
        h = []
        for b in range(2):
            h.append(pltpu.async_copy(
                src3.at[pl.ds(row0 + b * _GRP, _GRP)], idxd.at[b], si[b]))
        for b in range(2):
            h[b].wait()
            for k in range(_GRP):
                pltpu.async_copy(ones, acc.at[idxd.at[b, k]], ss[b], add=True)
        return _

    lax.fori_loop(0, niter, _iter, None)
    for b in range(2):
        for k in range(_GRP):
            pltpu.make_async_copy(ones, acc.at[idxd.at[b, k]], ss[b]).wait()
    plsc.subcore_barrier()
    pltpu.sync_copy(acc.at[pl.ds(r0, _RPT)], out.at[c].at[pl.ds(r0, _RPT)])


@jax.jit
def _degrees(src3):
    mesh = plsc.VectorSubcoreMesh(core_axis_name="c", subcore_axis_name="s")
    kern = pl.kernel(
        _deg_body,
        out_type=jax.ShapeDtypeStruct((_NC, _NPAD, 16), jnp.float32),
        mesh=mesh,
        compiler_params=pltpu.CompilerParams(use_tc_tiling_on_sc=False),
        scratch_types=[
            pltpu.VMEM((2, _GRP, _CH), jnp.int32),
            pltpu.VMEM((_CH, 16), jnp.float32),
            pltpu.VMEM((_ZR, 16), jnp.float32),
            pltpu.SemaphoreType.DMA,
            pltpu.SemaphoreType.DMA,
            pltpu.SemaphoreType.DMA,
            pltpu.SemaphoreType.DMA,
            pltpu.VMEM_SHARED((_NPAD, 16), jnp.float32),
        ],
    )
    return kern(src3)


def _mm_body(x_ref, w_ref, o_ref):
    o_ref[...] = jnp.dot(x_ref[...], w_ref[...],
                         preferred_element_type=jnp.float32)


def _matmul(x, w):
    m, k = x.shape
    _, n = w.shape
    bm = 512
    return pl.pallas_call(
        _mm_body,
        grid=(pl.cdiv(m, bm),),
        in_specs=[pl.BlockSpec((bm, k), lambda i: (i, 0)),
                  pl.BlockSpec((k, n), lambda i: (0, 0))],
        out_specs=pl.BlockSpec((bm, n), lambda i: (i, 0)),
        out_shape=jax.ShapeDtypeStruct((m, n), jnp.float32),
    )(x, w)


def _quarters(g, qw, width):
    g4 = g.reshape(_N, 4, qw).transpose(1, 0, 2)
    return jnp.pad(g4, ((0, 0), (0, _NPAD - _N), (0, width - qw)))


def _unquarters(s4, qw):
    return s4[:, :_N, :qw].transpose(1, 0, 2).reshape(_N, 4 * qw)


def kernel(x, edge_index, conv1_W, conv1_b, sk1_W, sk1_b, conv2_W, conv2_b,
           sk2_W, sk2_b, conv3_W, conv3_b, sk3_W, sk3_b, conv4_W, conv4_b,
           sk4_W, sk4_b, conv5_W, conv5_b, sk5_W, sk5_b, mix_W, mix_b,
           bn1_gamma, bn1_beta, bn1_mean, bn1_var,
           bn2_gamma, bn2_beta, bn2_mean, bn2_var,
           bn3_gamma, bn3_beta, bn3_mean, bn3_var,
           bn4_gamma, bn4_beta, bn4_mean, bn4_var,
           bn5_gamma, bn5_beta, bn5_mean, bn5_var):
    src = edge_index[0]
    dst = edge_index[1]

    npad = _EPAD - _E
    pad_src = (jnp.arange(npad, dtype=jnp.int32) * 97) % _N
    pad_dst = _N + (jnp.arange(npad, dtype=jnp.int32) % (_NPAD - _N))
    src_p = jnp.concatenate([src, pad_src])
    dst_p = jnp.concatenate([dst, pad_dst])
    srcq = jnp.stack([src_p + q * _NPAD for q in range(4)]
                     ).reshape(4, _EPAD // _CH, _CH)
    dst3 = dst_p.reshape(_EPAD // _CH, _CH)
    src3 = jnp.concatenate([src, pad_dst]).reshape(_EPAD // _CH, _CH)

    degp = _degrees(src3)
    deg = degp[0, :_N, 0] + degp[1, :_N, 0]
    dinv = jnp.where(deg > 0, lax.rsqrt(jnp.maximum(deg, 1.0)), 0.0)
    dinv = dinv[:, None]
    dinv2 = dinv * dinv

    def prop_q(tab, width):
        return _segsum(tab.reshape(4 * _NPAD, width), srcq, dst3, width)

    dinv2p = jnp.pad(dinv2, ((0, _NPAD - _N), (0, 0)))
    d2q = dinv2p[None, :, :]

    params = dict(
        conv=[(conv1_W, conv1_b), (conv2_W, conv2_b), (conv3_W, conv3_b),
              (conv4_W, conv4_b), (conv5_W, conv5_b)],
        sk=[(sk1_W, sk1_b), (sk2_W, sk2_b), (sk3_W, sk3_b), (sk4_W, sk4_b),
            (sk5_W, sk5_b)],
        bn=[(bn1_gamma, bn1_beta, bn1_mean, bn1_var),
            (bn2_gamma, bn2_beta, bn2_mean, bn2_var),
            (bn3_gamma, bn3_beta, bn3_mean, bn3_var),
            (bn4_gamma, bn4_beta, bn4_mean, bn4_var),
            (bn5_gamma, bn5_beta, bn5_mean, bn5_var)],
    )

    h = x
    for i in range(5):
        W, b = params["conv"][i]
        skW, skb = params["sk"][i]
        gam, bet, mean, var = params["bn"][i]
        A = jnp.concatenate([4.0 * W[3], 2.0 * W[2], W[1] - 3.0 * W[3],
                             W[0] - W[2], skW[0]], axis=1)
        Y = _matmul(h, A)
        Y3, Y2, Y1, Y0, Ysk = (Y[:, :80], Y[:, 80:160], Y[:, 160:240],
                               Y[:, 240:320], Y[:, 320:400])
        s4 = prop_q(_quarters(dinv * Y3, 20, 24), 24)
        s4 = prop_q(_quarters(dinv * Y2, 20, 24) - d2q * s4, 24)
        s4 = prop_q(_quarters(dinv * Y1, 20, 24) - d2q * s4, 24)
        pre = Y0 - dinv * _unquarters(s4, 20)
        scale = gam * lax.rsqrt(var + 1e-5)
        shift = bet - mean * scale + b * scale
        h = jax.nn.relu(pre * scale + shift) + Ysk + skb

    hc = jnp.concatenate([h, x], axis=1)
    Am = jnp.concatenate([mix_W[1], mix_W[0]], axis=1)
    Ym = _matmul(hc, Am)
    g = jnp.pad(dinv * Ym[:, :48], ((0, 0), (0, 16)))
    s4 = prop_q(_quarters(g, 16, 16), 16)
    s = _unquarters(s4, 16)[:, :48]
    return Ym[:, 48:96] + mix_b - dinv * s


# speedup vs baseline: 6.4190x; 1.0613x over previous
import functools

import jax
import jax.numpy as jnp
from jax import lax
from jax.experimental import pallas as pl
from jax.experimental.pallas import tpu as pltpu
from jax.experimental.pallas import tpu_sc as plsc

_N = 50000
_E = 800000
_NC, _NS = 2, 16
_CH = 320
_GRP = 2
_GE = _CH * _GRP
_EPAD = 819200
_NPAD = 50048
_RPT = _NPAD // _NS
_ZR = 136


def _seg_body(width, qw, tab, srcq, dst3, out, outw, idxs, idxd, rows, zbuf,
              si0, si1, sg0, sg1, ss0, ss1, acc):
    c = lax.axis_index("c")
    t = lax.axis_index("s")
    si = (si0, si1)
    sg = (sg0, sg1)
    ss = (ss0, ss1)

    def _zrow(r, _):
        for j0 in range(0, width - 15, 16):
            zbuf[r, j0:j0 + 16] = jnp.zeros((16,), jnp.float32)
        if width % 16:
            zbuf[r, width - 16:width] = jnp.zeros((16,), jnp.float32)
        return _
    lax.fori_loop(0, _ZR, _zrow, None)

    r0 = t * _RPT
    ept_rows = _EPAD // _NS // _CH
    niter = ept_rows // (2 * _GRP)

    for rnd in range(2):
        q = 2 * c + rnd
        for z in range(_RPT // _ZR):
            pltpu.async_copy(zbuf, acc.at[pl.ds(r0 + z * _ZR, _ZR)], sg0)
        for z in range(_RPT // _ZR):
            pltpu.make_async_copy(
                zbuf, acc.at[pl.ds(r0 + z * _ZR, _ZR)], sg0).wait()
        plsc.subcore_barrier()

        def _iter(i, _):
            @pl.when(i > 0)
            def _():
                for b in range(2):
                    for k in range(_GRP):
                        pltpu.make_async_copy(
                            rows.at[b, k], acc.at[idxd.at[b, k]], ss[b]).wait()

            row0 = t * ept_rows + i * 2 * _GRP
            hidx = []
            for b in range(2):
                hidx.append((
                    pltpu.async_copy(
                        srcq.at[q].at[pl.ds(row0 + b * _GRP, _GRP)],
                        idxs.at[b], si[b]),
                    pltpu.async_copy(
                        dst3.at[pl.ds(row0 + b * _GRP, _GRP)],
                        idxd.at[b], si[b]),
                ))
            hg = [[None] * _GRP for _ in range(2)]
            for b in range(2):
                for h in hidx[b]:
                    h.wait()
                for k in range(_GRP):
                    hg[b][k] = pltpu.async_copy(
                        tab.at[idxs.at[b, k]], rows.at[b, k], sg[b])
            for b in range(2):
                for k in range(_GRP):
                    hg[b][k].wait()
                    pltpu.async_copy(
                        rows.at[b, k], acc.at[idxd.at[b, k]], ss[b], add=True)
            return _

        lax.fori_loop(0, niter, _iter, None)
        for b in range(2):
            for k in range(_GRP):
                pltpu.make_async_copy(
                    rows.at[b, k], acc.at[idxd.at[b, k]], ss[b]).wait()
        plsc.subcore_barrier()
        pltpu.sync_copy(acc.at[pl.ds(r0, _RPT)], out.at[q].at[pl.ds(r0, _RPT)])
        pltpu.sync_copy(acc.at[pl.ds(r0, _RPT)],
                        outw.at[pl.ds(r0, _RPT), pl.ds(width * q, width)])
        plsc.subcore_barrier()


@functools.partial(jax.jit, static_argnums=(3, 4))
def _segsum(tab, srcq, dst3, width, qw):
    mesh = plsc.VectorSubcoreMesh(core_axis_name="c", subcore_axis_name="s")
    kern = pl.kernel(
        functools.partial(_seg_body, width, qw),
        out_type=(jax.ShapeDtypeStruct((4, _NPAD, width), jnp.float32),
                  jax.ShapeDtypeStruct((_NPAD, 128), jnp.float32)),
        mesh=mesh,
        compiler_params=pltpu.CompilerParams(use_tc_tiling_on_sc=False),
        scratch_types=[
            pltpu.VMEM((2, _GRP, _CH), jnp.int32),
            pltpu.VMEM((2, _GRP, _CH), jnp.int32),
            pltpu.VMEM((2, _GRP, _CH, width), jnp.float32),
            pltpu.VMEM((_ZR, width), jnp.float32),
            pltpu.SemaphoreType.DMA,
            pltpu.SemaphoreType.DMA,
            pltpu.SemaphoreType.DMA,
            pltpu.SemaphoreType.DMA,
            pltpu.SemaphoreType.DMA,
            pltpu.SemaphoreType.DMA,
            pltpu.VMEM_SHARED((_NPAD, width), jnp.float32),
        ],
    )
    return kern(tab, srcq, dst3)


def _deg_body(src3, out, idxd, ones, zbuf, si0, si1, ss0, ss1, acc):
    c = lax.axis_index("c")
    t = lax.axis_index("s")
    si = (si0, si1)
    ss = (ss0, ss1)

    def _orow(r, _):
        ones[r, 0:16] = jnp.ones((16,), jnp.float32)
        return _
    lax.fori_loop(0, _CH, _orow, None)

    def _zrow(r, _):
        zbuf[r, 0:16] = jnp.zeros((16,), jnp.float32)
        return _
    lax.fori_loop(0, _ZR, _zrow, None)
    r0 = t * _RPT
    for z in range(_RPT // _ZR):
        pltpu.async_copy(zbuf, acc.at[pl.ds(r0 + z * _ZR, _ZR)], ss0)
    for z in range(_RPT // _ZR):
        pltpu.make_async_copy(zbuf, acc.at[pl.ds(r0 + z * _ZR, _ZR)], ss0).wait()
    plsc.subcore_barrier()

    rows_per_tile = _EPAD // _CH // _NC // _NS
    niter = rows_per_tile // (2 * _GRP)

    def _iter(i, _):
        @pl.when(i > 0)
        def _():
            for b in range(2):
                for k in range(_GRP):
                    pltpu.make_async_copy(
                        ones, acc.at[idxd.at[b, k]], ss[b]).wait()
        row0 = (c * _NS + t) * rows_per_tile + i * 2 * _GRP
        h = []
        for b in range(2):
            h.append(pltpu.async_copy(
                src3.at[pl.ds(row0 + b * _GRP, _GRP)], idxd.at[b], si[b]))
        for b in range(2):
            h[b].wait()
            for k in range(_GRP):
                pltpu.async_copy(ones, acc.at[idxd.at[b, k]], ss[b], add=True)
        return _

    lax.fori_loop(0, niter, _iter, None)
    for b in range(2):
        for k in range(_GRP):
            pltpu.make_async_copy(ones, acc.at[idxd.at[b, k]], ss[b]).wait()
    plsc.subcore_barrier()
    pltpu.sync_copy(acc.at[pl.ds(r0, _RPT)], out.at[c].at[pl.ds(r0, _RPT)])


@jax.jit
def _degrees(src3):
    mesh = plsc.VectorSubcoreMesh(core_axis_name="c", subcore_axis_name="s")
    kern = pl.kernel(
        _deg_body,
        out_type=jax.ShapeDtypeStruct((_NC, _NPAD, 16), jnp.float32),
        mesh=mesh,
        compiler_params=pltpu.CompilerParams(use_tc_tiling_on_sc=False),
        scratch_types=[
            pltpu.VMEM((2, _GRP, _CH), jnp.int32),
            pltpu.VMEM((_CH, 16), jnp.float32),
            pltpu.VMEM((_ZR, 16), jnp.float32),
            pltpu.SemaphoreType.DMA,
            pltpu.SemaphoreType.DMA,
            pltpu.SemaphoreType.DMA,
            pltpu.SemaphoreType.DMA,
            pltpu.VMEM_SHARED((_NPAD, 16), jnp.float32),
        ],
    )
    return kern(src3)


def _mm_body(x_ref, w_ref, o_ref):
    o_ref[...] = jnp.dot(x_ref[...], w_ref[...],
                         preferred_element_type=jnp.float32)


def _matmul(x, w):
    m, k = x.shape
    _, n = w.shape
    bm = 512
    return pl.pallas_call(
        _mm_body,
        grid=(pl.cdiv(m, bm),),
        in_specs=[pl.BlockSpec((bm, k), lambda i: (i, 0)),
                  pl.BlockSpec((k, n), lambda i: (0, 0))],
        out_specs=pl.BlockSpec((bm, n), lambda i: (i, 0)),
        out_shape=jax.ShapeDtypeStruct((m, n), jnp.float32),
    )(x, w)


def _quarters(g, qw, width):
    g4 = g.reshape(_N, 4, qw).transpose(1, 0, 2)
    return jnp.pad(g4, ((0, 0), (0, _NPAD - _N), (0, width - qw)))


def _unquarters(s4, qw):
    return s4[:, :_N, :qw].transpose(1, 0, 2).reshape(_N, 4 * qw)


def kernel(x, edge_index, conv1_W, conv1_b, sk1_W, sk1_b, conv2_W, conv2_b,
           sk2_W, sk2_b, conv3_W, conv3_b, sk3_W, sk3_b, conv4_W, conv4_b,
           sk4_W, sk4_b, conv5_W, conv5_b, sk5_W, sk5_b, mix_W, mix_b,
           bn1_gamma, bn1_beta, bn1_mean, bn1_var,
           bn2_gamma, bn2_beta, bn2_mean, bn2_var,
           bn3_gamma, bn3_beta, bn3_mean, bn3_var,
           bn4_gamma, bn4_beta, bn4_mean, bn4_var,
           bn5_gamma, bn5_beta, bn5_mean, bn5_var):
    src = edge_index[0]
    dst = edge_index[1]

    npad = _EPAD - _E
    pad_src = (jnp.arange(npad, dtype=jnp.int32) * 97) % _N
    pad_dst = _N + (jnp.arange(npad, dtype=jnp.int32) % (_NPAD - _N))
    src_p = jnp.concatenate([src, pad_src])
    dst_p = jnp.concatenate([dst, pad_dst])
    srcq = jnp.stack([src_p + q * _NPAD for q in range(4)]
                     ).reshape(4, _EPAD // _CH, _CH)
    dst3 = dst_p.reshape(_EPAD // _CH, _CH)
    src3 = jnp.concatenate([src, pad_dst]).reshape(_EPAD // _CH, _CH)

    degp = _degrees(src3)
    deg = degp[0, :_N, 0] + degp[1, :_N, 0]
    dinv = jnp.where(deg > 0, lax.rsqrt(jnp.maximum(deg, 1.0)), 0.0)
    dinv = dinv[:, None]
    dinv2 = dinv * dinv

    def prop_q(tab, width, qw):
        return _segsum(tab.reshape(4 * _NPAD, width), srcq, dst3, width, qw)

    dinv2p = jnp.pad(dinv2, ((0, _NPAD - _N), (0, 0)))
    d2q = dinv2p[None, :, :]

    params = dict(
        conv=[(conv1_W, conv1_b), (conv2_W, conv2_b), (conv3_W, conv3_b),
              (conv4_W, conv4_b), (conv5_W, conv5_b)],
        sk=[(sk1_W, sk1_b), (sk2_W, sk2_b), (sk3_W, sk3_b), (sk4_W, sk4_b),
            (sk5_W, sk5_b)],
        bn=[(bn1_gamma, bn1_beta, bn1_mean, bn1_var),
            (bn2_gamma, bn2_beta, bn2_mean, bn2_var),
            (bn3_gamma, bn3_beta, bn3_mean, bn3_var),
            (bn4_gamma, bn4_beta, bn4_mean, bn4_var),
            (bn5_gamma, bn5_beta, bn5_mean, bn5_var)],
    )

    h = x
    for i in range(5):
        W, b = params["conv"][i]
        skW, skb = params["sk"][i]
        gam, bet, mean, var = params["bn"][i]
        A = jnp.concatenate([4.0 * W[3], 2.0 * W[2], W[1] - 3.0 * W[3],
                             W[0] - W[2], skW[0]], axis=1)
        Y = _matmul(h, A)
        Y3, Y2, Y1, Y0, Ysk = (Y[:, :80], Y[:, 80:160], Y[:, 160:240],
                               Y[:, 240:320], Y[:, 320:400])
        s4, _ = prop_q(_quarters(dinv * Y3, 20, 24), 24, 20)
        s4, _ = prop_q(_quarters(dinv * Y2, 20, 24) - d2q * s4, 24, 20)
        _, sw = prop_q(_quarters(dinv * Y1, 20, 24) - d2q * s4, 24, 20)
        s = jnp.concatenate([sw[:_N, 24 * q:24 * q + 20] for q in range(4)],
                            axis=1)
        pre = Y0 - dinv * s
        scale = gam * lax.rsqrt(var + 1e-5)
        shift = bet - mean * scale + b * scale
        h = jax.nn.relu(pre * scale + shift) + Ysk + skb

    hc = jnp.concatenate([h, x], axis=1)
    Am = jnp.concatenate([mix_W[1], mix_W[0]], axis=1)
    Ym = _matmul(hc, Am)
    g = jnp.pad(dinv * Ym[:, :48], ((0, 0), (0, 16)))
    _, sw = prop_q(_quarters(g, 16, 16), 16, 16)
    return Ym[:, 48:96] + mix_b - dinv * sw[:_N, :48]


# speedup vs baseline: 6.5087x; 1.0140x over previous
import functools

import jax
import jax.numpy as jnp
from jax import lax
from jax.experimental import pallas as pl
from jax.experimental.pallas import tpu as pltpu
from jax.experimental.pallas import tpu_sc as plsc

_N = 50000
_E = 800000
_NC, _NS = 2, 16
_CH = 512
_EPAD = 819200
_GPT = _EPAD // _NS // _CH
_NPAD = 50048
_RPT = _NPAD // _NS
_ZR = 136


def _seg_body(width, qw, tab, srcq, dst3, out, outw, idxs, idxd, rows, zbuf,
              si0, si1, si2, sg0, sg1, sg2, ss0, ss1, ss2, acc):
    c = lax.axis_index("c")
    t = lax.axis_index("s")
    si = (si0, si1, si2)
    sg = (sg0, sg1, sg2)
    ss = (ss0, ss1, ss2)

    def _zrow(r, _):
        for j0 in range(0, width - 15, 16):
            zbuf[r, j0:j0 + 16] = jnp.zeros((16,), jnp.float32)
        if width % 16:
            zbuf[r, width - 16:width] = jnp.zeros((16,), jnp.float32)
        return _
    lax.fori_loop(0, _ZR, _zrow, None)

    r0 = t * _RPT

    def _load_idx(q, k, b):
        row = t * _GPT + k
        pltpu.async_copy(srcq.at[q, row], idxs.at[b], si[b])
        pltpu.async_copy(dst3.at[row], idxd.at[b], si[b])

    def _wait_idx(b):
        pltpu.make_async_copy(srcq.at[0, 0], idxs.at[b], si[b]).wait()
        pltpu.make_async_copy(dst3.at[0], idxd.at[b], si[b]).wait()

    def _scatter(b):
        pltpu.make_async_copy(tab.at[idxs.at[b]], rows.at[b], sg[b]).wait()
        pltpu.async_copy(rows.at[b], acc.at[idxd.at[b]], ss[b], add=True)

    def _drain_sc(b):
        pltpu.make_async_copy(rows.at[b], acc.at[idxd.at[b]], ss[b]).wait()

    for rnd in range(2):
        q = 2 * c + rnd
        for z in range(_RPT // _ZR):
            pltpu.async_copy(zbuf, acc.at[pl.ds(r0 + z * _ZR, _ZR)], sg0)
        for z in range(_RPT // _ZR):
            pltpu.make_async_copy(
                zbuf, acc.at[pl.ds(r0 + z * _ZR, _ZR)], sg0).wait()
        plsc.subcore_barrier()

        _load_idx(q, 0, 0)
        _load_idx(q, 1, 1)
        _wait_idx(0)
        pltpu.async_copy(tab.at[idxs.at[0]], rows.at[0], sg[0])

        def _step(i, _):
            for j in range(3):
                k = 3 * i + 1 + j
                b = (1 + j) % 3
                bp = j % 3
                bn = (2 + j) % 3
                _scatter(bp)
                @pl.when(k <= _GPT - 2)
                def _():
                    @pl.when(k >= 2)
                    def _():
                        _drain_sc(bn)
                    _load_idx(q, k + 1, bn)
                @pl.when(k <= _GPT - 1)
                def _():
                    _wait_idx(b)
                    pltpu.async_copy(tab.at[idxs.at[b]], rows.at[b], sg[b])
            return _

        lax.fori_loop(0, (_GPT - 1) // 3, _step, None)
        _scatter((_GPT - 1) % 3)
        for b in range(3):
            _drain_sc(b)
        plsc.subcore_barrier()
        pltpu.sync_copy(acc.at[pl.ds(r0, _RPT)], out.at[q].at[pl.ds(r0, _RPT)])
        pltpu.sync_copy(acc.at[pl.ds(r0, _RPT)],
                        outw.at[pl.ds(r0, _RPT), pl.ds(width * q, width)])
        plsc.subcore_barrier()


@functools.partial(jax.jit, static_argnums=(3, 4))
def _segsum(tab, srcq, dst3, width, qw):
    mesh = plsc.VectorSubcoreMesh(core_axis_name="c", subcore_axis_name="s")
    kern = pl.kernel(
        functools.partial(_seg_body, width, qw),
        out_type=(jax.ShapeDtypeStruct((4, _NPAD, width), jnp.float32),
                  jax.ShapeDtypeStruct((_NPAD, 128), jnp.float32)),
        mesh=mesh,
        compiler_params=pltpu.CompilerParams(use_tc_tiling_on_sc=False),
        scratch_types=[
            pltpu.VMEM((3, _CH), jnp.int32),
            pltpu.VMEM((3, _CH), jnp.int32),
            pltpu.VMEM((3, _CH, width), jnp.float32),
            pltpu.VMEM((_ZR, width), jnp.float32),
            pltpu.SemaphoreType.DMA,
            pltpu.SemaphoreType.DMA,
            pltpu.SemaphoreType.DMA,
            pltpu.SemaphoreType.DMA,
            pltpu.SemaphoreType.DMA,
            pltpu.SemaphoreType.DMA,
            pltpu.SemaphoreType.DMA,
            pltpu.SemaphoreType.DMA,
            pltpu.SemaphoreType.DMA,
            pltpu.VMEM_SHARED((_NPAD, width), jnp.float32),
        ],
    )
    return kern(tab, srcq, dst3)


def _deg_body(src3, out, idxd, ones, zbuf, si0, si1, ss0, ss1, acc):
    c = lax.axis_index("c")
    t = lax.axis_index("s")
    si = (si0, si1)
    ss = (ss0, ss1)

    def _orow(r, _):
        ones[r, 0:16] = jnp.ones((16,), jnp.float32)
        return _
    lax.fori_loop(0, _CH, _orow, None)

    def _zrow(r, _):
        zbuf[r, 0:16] = jnp.zeros((16,), jnp.float32)
        return _
    lax.fori_loop(0, _ZR, _zrow, None)
    r0 = t * _RPT
    for z in range(_RPT // _ZR):
        pltpu.async_copy(zbuf, acc.at[pl.ds(r0 + z * _ZR, _ZR)], ss0)
    for z in range(_RPT // _ZR):
        pltpu.make_async_copy(zbuf, acc.at[pl.ds(r0 + z * _ZR, _ZR)], ss0).wait()
    plsc.subcore_barrier()

    rows_per_tile = _EPAD // _CH // _NC // _NS
    niter = rows_per_tile // 2

    def _iter(i, _):
        @pl.when(i > 0)
        def _():
            for b in range(2):
                pltpu.make_async_copy(ones, acc.at[idxd.at[b]], ss[b]).wait()
        row0 = (c * _NS + t) * rows_per_tile + i * 2
        h = []
        for b in range(2):
            h.append(pltpu.async_copy(src3.at[row0 + b], idxd.at[b], si[b]))
        for b in range(2):
            h[b].wait()
            pltpu.async_copy(ones, acc.at[idxd.at[b]], ss[b], add=True)
        return _

    lax.fori_loop(0, niter, _iter, None)
    for b in range(2):
        pltpu.make_async_copy(ones, acc.at[idxd.at[b]], ss[b]).wait()
    plsc.subcore_barrier()
    pltpu.sync_copy(acc.at[pl.ds(r0, _RPT)], out.at[c].at[pl.ds(r0, _RPT)])


@jax.jit
def _degrees(src3):
    mesh = plsc.VectorSubcoreMesh(core_axis_name="c", subcore_axis_name="s")
    kern = pl.kernel(
        _deg_body,
        out_type=jax.ShapeDtypeStruct((_NC, _NPAD, 16), jnp.float32),
        mesh=mesh,
        compiler_params=pltpu.CompilerParams(use_tc_tiling_on_sc=False),
        scratch_types=[
            pltpu.VMEM((2, _CH), jnp.int32),
            pltpu.VMEM((_CH, 16), jnp.float32),
            pltpu.VMEM((_ZR, 16), jnp.float32),
            pltpu.SemaphoreType.DMA,
            pltpu.SemaphoreType.DMA,
            pltpu.SemaphoreType.DMA,
            pltpu.SemaphoreType.DMA,
            pltpu.VMEM_SHARED((_NPAD, 16), jnp.float32),
        ],
    )
    return kern(src3)


def _mm_body(x_ref, w_ref, o_ref):
    o_ref[...] = jnp.dot(x_ref[...], w_ref[...],
                         preferred_element_type=jnp.float32)


def _matmul(x, w):
    m, k = x.shape
    _, n = w.shape
    bm = 512
    return pl.pallas_call(
        _mm_body,
        grid=(pl.cdiv(m, bm),),
        in_specs=[pl.BlockSpec((bm, k), lambda i: (i, 0)),
                  pl.BlockSpec((k, n), lambda i: (0, 0))],
        out_specs=pl.BlockSpec((bm, n), lambda i: (i, 0)),
        out_shape=jax.ShapeDtypeStruct((m, n), jnp.float32),
    )(x, w)


def _quarters(g, qw, width):
    g4 = g.reshape(_N, 4, qw).transpose(1, 0, 2)
    return jnp.pad(g4, ((0, 0), (0, _NPAD - _N), (0, width - qw)))


def _unquarters(s4, qw):
    return s4[:, :_N, :qw].transpose(1, 0, 2).reshape(_N, 4 * qw)


def kernel(x, edge_index, conv1_W, conv1_b, sk1_W, sk1_b, conv2_W, conv2_b,
           sk2_W, sk2_b, conv3_W, conv3_b, sk3_W, sk3_b, conv4_W, conv4_b,
           sk4_W, sk4_b, conv5_W, conv5_b, sk5_W, sk5_b, mix_W, mix_b,
           bn1_gamma, bn1_beta, bn1_mean, bn1_var,
           bn2_gamma, bn2_beta, bn2_mean, bn2_var,
           bn3_gamma, bn3_beta, bn3_mean, bn3_var,
           bn4_gamma, bn4_beta, bn4_mean, bn4_var,
           bn5_gamma, bn5_beta, bn5_mean, bn5_var):
    src = edge_index[0]
    dst = edge_index[1]

    npad = _EPAD - _E
    pad_src = (jnp.arange(npad, dtype=jnp.int32) * 97) % _N
    pad_dst = _N + (jnp.arange(npad, dtype=jnp.int32) % (_NPAD - _N))
    src_p = jnp.concatenate([src, pad_src])
    dst_p = jnp.concatenate([dst, pad_dst])
    srcq = jnp.stack([src_p + q * _NPAD for q in range(4)]
                     ).reshape(4, _EPAD // _CH, _CH)
    dst3 = dst_p.reshape(_EPAD // _CH, _CH)
    src3 = jnp.concatenate([src, pad_dst]).reshape(_EPAD // _CH, _CH)

    degp = _degrees(src3)
    deg = degp[0, :_N, 0] + degp[1, :_N, 0]
    dinv = jnp.where(deg > 0, lax.rsqrt(jnp.maximum(deg, 1.0)), 0.0)
    dinv = dinv[:, None]
    dinv2 = dinv * dinv

    def prop_q(tab, width, qw):
        return _segsum(tab.reshape(4 * _NPAD, width), srcq, dst3, width, qw)

    dinv2p = jnp.pad(dinv2, ((0, _NPAD - _N), (0, 0)))
    d2q = dinv2p[None, :, :]

    params = dict(
        conv=[(conv1_W, conv1_b), (conv2_W, conv2_b), (conv3_W, conv3_b),
              (conv4_W, conv4_b), (conv5_W, conv5_b)],
        sk=[(sk1_W, sk1_b), (sk2_W, sk2_b), (sk3_W, sk3_b), (sk4_W, sk4_b),
            (sk5_W, sk5_b)],
        bn=[(bn1_gamma, bn1_beta, bn1_mean, bn1_var),
            (bn2_gamma, bn2_beta, bn2_mean, bn2_var),
            (bn3_gamma, bn3_beta, bn3_mean, bn3_var),
            (bn4_gamma, bn4_beta, bn4_mean, bn4_var),
            (bn5_gamma, bn5_beta, bn5_mean, bn5_var)],
    )

    h = x
    for i in range(5):
        W, b = params["conv"][i]
        skW, skb = params["sk"][i]
        gam, bet, mean, var = params["bn"][i]
        A = jnp.concatenate([4.0 * W[3], 2.0 * W[2], W[1] - 3.0 * W[3],
                             W[0] - W[2], skW[0]], axis=1)
        Y = _matmul(h, A)
        Y3, Y2, Y1, Y0, Ysk = (Y[:, :80], Y[:, 80:160], Y[:, 160:240],
                               Y[:, 240:320], Y[:, 320:400])
        s4, _ = prop_q(_quarters(dinv * Y3, 20, 24), 24, 20)
        s4, _ = prop_q(_quarters(dinv * Y2, 20, 24) - d2q * s4, 24, 20)
        _, sw = prop_q(_quarters(dinv * Y1, 20, 24) - d2q * s4, 24, 20)
        s = jnp.concatenate([sw[:_N, 24 * q:24 * q + 20] for q in range(4)],
                            axis=1)
        pre = Y0 - dinv * s
        scale = gam * lax.rsqrt(var + 1e-5)
        shift = bet - mean * scale + b * scale
        h = jax.nn.relu(pre * scale + shift) + Ysk + skb

    hc = jnp.concatenate([h, x], axis=1)
    Am = jnp.concatenate([mix_W[1], mix_W[0]], axis=1)
    Ym = _matmul(hc, Am)
    g = jnp.pad(dinv * Ym[:, :48], ((0, 0), (0, 16)))
    _, sw = prop_q(_quarters(g, 16, 16), 16, 16)
    return Ym[:, 48:96] + mix_b - dinv * sw[:_N, :48]
